# Initial kernel scaffold; baseline (speedup 1.0000x reference)
import jax
import jax.numpy as jnp
from jax.experimental import pallas as pl


def kernel(input_ids, weight):
    raise NotImplementedError("write your pallas kernel here")


# speedup vs baseline: 3.1149x; 3.1149x over previous
import jax
import jax.numpy as jnp
from jax.experimental import pallas as pl
from jax.experimental.pallas import tpu as pltpu
from jax.experimental.pallas import tpu_sc as plsc

EMBED_DIM = 128
WINDOW = 128


def _gather_sc(weight, flat_ids):
    num_idx = flat_ids.shape[1]
    mesh = plsc.VectorSubcoreMesh(core_axis_name="core", subcore_axis_name="subcore")

    @pl.kernel(
        out_type=jax.ShapeDtypeStruct((num_idx, EMBED_DIM), weight.dtype),
        mesh=mesh,
    )
    def k(w_hbm, ids_hbm, out_hbm):
        def body(i_vmem, o_vmem):
            pltpu.sync_copy(w_hbm.at[i_vmem.at[0]], o_vmem)

        pltpu.emit_pipeline(
            body,
            grid=(num_idx // WINDOW,),
            in_specs=[pl.BlockSpec((1, WINDOW), index_map=lambda i: (0, i))],
            out_specs=[pl.BlockSpec((WINDOW, EMBED_DIM), index_map=lambda i: (i, 0))],
            core_axis_name=("core", "subcore"),
            dimension_semantics=(pltpu.PARALLEL,),
        )(ids_hbm, out_hbm)

    return k(weight, flat_ids)


def kernel(input_ids, weight):
    batch, seq_len = input_ids.shape
    flat = input_ids.reshape(1, batch * seq_len).astype(jnp.int32)
    out = _gather_sc(weight, flat)
    return out.reshape(batch, seq_len, EMBED_DIM)


# speedup vs baseline: 3.3108x; 1.0629x over previous
import jax
import jax.numpy as jnp
from jax.experimental import pallas as pl
from jax.experimental.pallas import tpu as pltpu
from jax.experimental.pallas import tpu_sc as plsc

EMBED_DIM = 128
WINDOW = 256


def _gather_sc(weight, flat_ids):
    num_idx = flat_ids.shape[1]
    mesh = plsc.VectorSubcoreMesh(core_axis_name="core", subcore_axis_name="subcore")

    @pl.kernel(
        out_type=jax.ShapeDtypeStruct((num_idx, EMBED_DIM), weight.dtype),
        mesh=mesh,
    )
    def k(w_hbm, ids_hbm, out_hbm):
        def body(i_vmem, o_vmem):
            pltpu.sync_copy(w_hbm.at[i_vmem.at[0]], o_vmem)

        pltpu.emit_pipeline(
            body,
            grid=(num_idx // WINDOW,),
            in_specs=[pl.BlockSpec((1, WINDOW), index_map=lambda i: (0, i))],
            out_specs=[pl.BlockSpec((WINDOW, EMBED_DIM), index_map=lambda i: (i, 0))],
            core_axis_name=("core", "subcore"),
            dimension_semantics=(pltpu.PARALLEL,),
        )(ids_hbm, out_hbm)

    return k(weight, flat_ids)


def kernel(input_ids, weight):
    batch, seq_len = input_ids.shape
    flat = input_ids.reshape(1, batch * seq_len).astype(jnp.int32)
    out = _gather_sc(weight, flat)
    return out.reshape(batch, seq_len, EMBED_DIM)


# speedup vs baseline: 5.7383x; 1.7332x over previous
import jax
import jax.numpy as jnp
from jax.experimental import pallas as pl
from jax.experimental.pallas import tpu as pltpu
from jax.experimental.pallas import tpu_sc as plsc

EMBED_DIM = 128
BATCH_BLK = 4


def _gather_sc(weight, ids3, batch, seq_len):
    num_steps = ids3.shape[0]
    win = ids3.shape[2]
    mesh = plsc.VectorSubcoreMesh(core_axis_name="core", subcore_axis_name="subcore")

    @pl.kernel(
        out_type=jax.ShapeDtypeStruct((batch, seq_len, EMBED_DIM), weight.dtype),
        mesh=mesh,
    )
    def k(w_hbm, ids_hbm, out_hbm):
        def body(i_vmem, o_vmem):
            o_rows = o_vmem.reshape(win, EMBED_DIM)
            pltpu.sync_copy(w_hbm.at[i_vmem.at[0, 0]], o_rows)

        pltpu.emit_pipeline(
            body,
            grid=(num_steps,),
            in_specs=[pl.BlockSpec((1, 1, win), index_map=lambda i: (i, 0, 0))],
            out_specs=[
                pl.BlockSpec((BATCH_BLK, seq_len, EMBED_DIM), index_map=lambda i: (i, 0, 0))
            ],
            core_axis_name=("core", "subcore"),
            dimension_semantics=(pltpu.PARALLEL,),
        )(ids_hbm, out_hbm)

    return k(weight, ids3)


def kernel(input_ids, weight):
    batch, seq_len = input_ids.shape
    win = BATCH_BLK * seq_len
    ids3 = input_ids.reshape(batch // BATCH_BLK, 1, win).astype(jnp.int32)
    return _gather_sc(weight, ids3, batch, seq_len)
